# Initial kernel scaffold; baseline (speedup 1.0000x reference)
import jax
import jax.numpy as jnp
from jax.experimental import pallas as pl


def kernel(feat_angle_dist_matrix, positive_masks, true_angle_dist_matrix):
    raise NotImplementedError("write your pallas kernel here")


# speedup vs baseline: 49.1373x; 49.1373x over previous
import jax
import jax.numpy as jnp
from jax.experimental import pallas as pl
from jax.experimental.pallas import tpu as pltpu

_EPS = 1e-6


def _angle_loss_kernel(n: int, feat_ref, true_ref, mask_ref, out_ref):
    p = n * n
    t_count = float(n * (n - 1) * (n - 2))

    f = feat_ref[:]
    u = true_ref[:]
    fn = jnp.sqrt(jnp.sum(f * f, axis=1, keepdims=True))
    fh = f / jnp.maximum(fn, _EPS)
    un = jnp.sqrt(jnp.sum(u * u, axis=1, keepdims=True))
    uh = u / jnp.maximum(un, _EPS)

    dn = (((1,), (1,)), ((), ()))
    a = jax.lax.dot_general(fh, fh, dn, precision=jax.lax.Precision.HIGHEST,
                            preferred_element_type=jnp.float32)
    b = jax.lax.dot_general(uh, uh, dn, precision=jax.lax.Precision.HIGHEST,
                            preferred_element_type=jnp.float32)

    r = jax.lax.broadcasted_iota(jnp.int32, (p, p), 0)
    c = jax.lax.broadcasted_iota(jnp.int32, (p, p), 1)
    ri, rj = r // n, r % n
    ci, ck = c // n, c % n
    valid = (ri == ci) & (rj != ri) & (ck != ci) & (rj != ck)
    vf = valid.astype(jnp.float32)

    s1 = jnp.sum(b * vf)
    s2 = jnp.sum(b * b * vf)
    ae = a + _EPS
    q = t_count * ae * ae - 2.0 * ae * s1 + s2
    total = jnp.sum(jnp.sqrt(jnp.maximum(q, 0.0)) * vf)

    m = mask_ref[:]
    mi = jax.lax.broadcasted_iota(jnp.int32, (n, n), 0)
    mj = jax.lax.broadcasted_iota(jnp.int32, (n, n), 1)
    gate = jnp.min(jnp.where(mi == mj, 1.0, m))

    out_ref[0, 0] = total / t_count * gate * 0.5


def kernel(feat_angle_dist_matrix, positive_masks, true_angle_dist_matrix):
    n = positive_masks.shape[0]
    d = feat_angle_dist_matrix.shape[-1]
    f = feat_angle_dist_matrix.reshape(n * n, d)
    u = true_angle_dist_matrix.reshape(n * n, d)
    m = positive_masks.astype(jnp.float32)

    import functools
    out = pl.pallas_call(
        functools.partial(_angle_loss_kernel, n),
        out_shape=jax.ShapeDtypeStruct((1, 1), jnp.float32),
        out_specs=pl.BlockSpec(memory_space=pltpu.SMEM),
    )(f, u, m)
    return out[0, 0]


# speedup vs baseline: 57.9479x; 1.1793x over previous
import functools

import jax
import jax.numpy as jnp
from jax.experimental import pallas as pl
from jax.experimental.pallas import tpu as pltpu

_EPS = 1e-6


def _angle_loss_kernel(n: int, feat_ref, true_ref, mask_ref, out_ref):
    t_count = float(n * (n - 1) * (n - 2))

    f = feat_ref[:]
    u = true_ref[:]
    fn = jnp.sqrt(jnp.sum(f * f, axis=-1, keepdims=True))
    fh = f / jnp.maximum(fn, _EPS)
    un = jnp.sqrt(jnp.sum(u * u, axis=-1, keepdims=True))
    uh = u / jnp.maximum(un, _EPS)

    dn = (((2,), (2,)), ((0,), (0,)))
    a = jax.lax.dot_general(fh, fh, dn, preferred_element_type=jnp.float32)
    b = jax.lax.dot_general(uh, uh, dn, preferred_element_type=jnp.float32)

    ii = jax.lax.broadcasted_iota(jnp.int32, (n, n, n), 0)
    jj = jax.lax.broadcasted_iota(jnp.int32, (n, n, n), 1)
    kk = jax.lax.broadcasted_iota(jnp.int32, (n, n, n), 2)
    valid = (jj != ii) & (kk != ii) & (jj != kk)
    vf = valid.astype(jnp.float32)

    s1 = jnp.sum(b * vf)
    s2 = jnp.sum(b * b * vf)
    ae = a + _EPS
    q = t_count * ae * ae - 2.0 * ae * s1 + s2
    total = jnp.sum(jnp.sqrt(jnp.maximum(q, 0.0)) * vf)

    m = mask_ref[:].astype(jnp.float32)
    mi = jax.lax.broadcasted_iota(jnp.int32, (n, n), 0)
    mj = jax.lax.broadcasted_iota(jnp.int32, (n, n), 1)
    gate = jnp.min(jnp.where(mi == mj, 1.0, m))

    out_ref[0, 0] = total / t_count * gate * 0.5


def kernel(feat_angle_dist_matrix, positive_masks, true_angle_dist_matrix):
    n = positive_masks.shape[0]
    out = pl.pallas_call(
        functools.partial(_angle_loss_kernel, n),
        out_shape=jax.ShapeDtypeStruct((1, 1), jnp.float32),
        out_specs=pl.BlockSpec(memory_space=pltpu.SMEM),
    )(feat_angle_dist_matrix, true_angle_dist_matrix, positive_masks)
    return out.reshape(())
